# baseline (speedup 1.0000x reference)
import functools

import jax
import jax.numpy as jnp
from jax import lax
from jax.experimental import pallas as pl
from jax.experimental.pallas import tpu as pltpu
from jax.experimental.pallas import tpu_sc as plsc

N = 100000
N_NODES = 10000
NUM_GRAPHS = 10
E = 3200000
D_IN = 9

NPAD = 100096
NGP = 10240
NG = NUM_GRAPHS * NGP
NW = 32
STRIPE = NPAD // 16
STRIPE_A = NG // 16
CHUNK = 2000
EPW_CHUNKS = E // (NW * CHUNK)

_PLB = 3328
_PGRID = -(-N // _PLB)


def _prologue_body(xt_ref, bt_ref, projt_ref, pmax_ref):
    xt = xt_ref[...]
    bt = bt_ref[...]
    r = lax.dot_general(bt, xt, (((1,), (0,)), ((), ())),
                        preferred_element_type=jnp.float32)
    projt_ref[...] = r
    col = lax.broadcasted_iota(jnp.int32, r.shape, 1) + pl.program_id(0) * _PLB
    rm = jnp.where(col < N, r, -3.4e38)
    pmax_ref[...] = jnp.max(rm, axis=1).reshape(1, 1, 3)


def _prologue(xt, btmat):
    return pl.pallas_call(
        _prologue_body,
        grid=(_PGRID,),
        in_specs=[
            pl.BlockSpec((D_IN, _PLB), lambda g: (0, g)),
            pl.BlockSpec((3, D_IN), lambda g: (0, 0)),
        ],
        out_specs=[
            pl.BlockSpec((3, _PLB), lambda g: (0, g)),
            pl.BlockSpec((1, 1, 3), lambda g: (g, 0, 0)),
        ],
        out_shape=[
            jax.ShapeDtypeStruct((3, NPAD), jnp.float32),
            jax.ShapeDtypeStruct((_PGRID, 1, 3), jnp.float32),
        ],
    )(xt, btmat)


def _sc_edge_kernel(e1, e2, p1f, p2f, mvec1, mvec2, zeros):
    mesh = plsc.VectorSubcoreMesh(core_axis_name="c", subcore_axis_name="s")

    out_type = [jax.ShapeDtypeStruct((2 * NG,), jnp.float32)] * 4
    scratch = [
        pltpu.VMEM_SHARED((NPAD,), jnp.float32),
        pltpu.VMEM_SHARED((NPAD,), jnp.float32),
        pltpu.VMEM_SHARED((NPAD,), jnp.float32),
        pltpu.VMEM_SHARED((NPAD,), jnp.float32),
        pltpu.VMEM_SHARED((NPAD,), jnp.float32),
        pltpu.VMEM_SHARED((NPAD,), jnp.float32),
        pltpu.VMEM_SHARED((NG,), jnp.float32),
        pltpu.VMEM_SHARED((NG,), jnp.float32),
        pltpu.VMEM_SHARED((NG,), jnp.float32),
        pltpu.VMEM_SHARED((NG,), jnp.float32),
        pltpu.VMEM((CHUNK,), jnp.int32),
        pltpu.VMEM((CHUNK,), jnp.int32),
        pltpu.VMEM((CHUNK,), jnp.int32),
        pltpu.VMEM((CHUNK,), jnp.int32),
        pltpu.VMEM((CHUNK,), jnp.int32),
        pltpu.VMEM((CHUNK,), jnp.float32),
        pltpu.VMEM((CHUNK,), jnp.float32),
        pltpu.VMEM((CHUNK,), jnp.float32),
        pltpu.VMEM((CHUNK,), jnp.float32),
        pltpu.VMEM((CHUNK,), jnp.float32),
        pltpu.VMEM((CHUNK,), jnp.float32),
        pltpu.VMEM((CHUNK,), jnp.float32),
        pltpu.VMEM((CHUNK,), jnp.float32),
        pltpu.VMEM((16,), jnp.float32),
        pltpu.VMEM((16,), jnp.float32),
        pltpu.VMEM((STRIPE_A,), jnp.float32),
        pltpu.SemaphoreType.DMA,
        pltpu.SemaphoreType.DMA,
        pltpu.SemaphoreType.DMA,
        pltpu.SemaphoreType.DMA,
        pltpu.SemaphoreType.DMA,
        pltpu.SemaphoreType.DMA,
    ]

    @functools.partial(pl.kernel, out_type=out_type, mesh=mesh,
                       scratch_types=scratch)
    def k(e1_h, e2_h, p1_h, p2_h, m1_h, m2_h, zeros_h,
          oden1, onum1, oden2, onum2,
          sp_a1, sp_d1, sp_s1, sp_a2, sp_d2, sp_s2,
          acc_d1, acc_n1, acc_d2, acc_n2,
          v_src0, v_src1, v_dst0, v_dst1, v_dstp,
          v_ga0, v_ga1, v_gs0, v_gs1, v_gd0, v_gd1, v_p, v_ps,
          v_m1, v_m2, v_stage,
          sem_a0, sem_a1, sem_s0, sem_s1, sem_d0, sem_d1):
        c = lax.axis_index("c")
        s = lax.axis_index("s")
        wid = s * 2 + c
        off = s * STRIPE
        offa = s * STRIPE_A
        sl = pl.ds(off, STRIPE)
        sla = pl.ds(offa, STRIPE_A)

        for t, sp in ((0, sp_a1), (1, sp_d1), (2, sp_s1)):
            pltpu.sync_copy(p1_h.at[pl.ds(t * NPAD + off, STRIPE)],
                            v_stage.at[pl.ds(0, STRIPE)])
            pltpu.sync_copy(v_stage.at[pl.ds(0, STRIPE)], sp.at[sl])
        for t, sp in ((0, sp_a2), (1, sp_d2), (2, sp_s2)):
            pltpu.sync_copy(p2_h.at[pl.ds(t * NPAD + off, STRIPE)],
                            v_stage.at[pl.ds(0, STRIPE)])
            pltpu.sync_copy(v_stage.at[pl.ds(0, STRIPE)], sp.at[sl])
        pltpu.sync_copy(zeros_h.at[sla], v_stage)
        for acc in (acc_d1, acc_n1, acc_d2, acc_n2):
            pltpu.sync_copy(v_stage, acc.at[sla])
        pltpu.sync_copy(m1_h, v_m1)
        pltpu.sync_copy(m2_h, v_m2)
        plsc.subcore_barrier()

        m1 = v_m1[...]
        m2 = v_m2[...]

        vsrc = (v_src0, v_src1)
        vdst = (v_dst0, v_dst1)
        vga = (v_ga0, v_ga1)
        vgs = (v_gs0, v_gs1)
        vgd = (v_gd0, v_gd1)
        sma = (sem_a0, sem_a1)
        sms = (sem_s0, sem_s1)
        smd = (sem_d0, sem_d1)

        def do_conv(e_h, sp_a, sp_d, sp_s, acc_d, acc_n, mv):
            def issue(par, kk):
                base = (wid * EPW_CHUNKS + kk) * CHUNK
                pltpu.sync_copy(e_h.at[pl.ds(base, CHUNK)], vsrc[par])
                pltpu.sync_copy(e_h.at[pl.ds(E + base, CHUNK)], vdst[par])
                pltpu.async_copy(sp_a.at[vsrc[par]], vga[par], sma[par])
                pltpu.async_copy(sp_s.at[vsrc[par]], vgs[par], sms[par])
                pltpu.async_copy(sp_d.at[vdst[par]], vgd[par], smd[par])

            def drain(par):
                pltpu.make_async_copy(sp_a.at[vsrc[par]], vga[par],
                                      sma[par]).wait()
                pltpu.make_async_copy(sp_s.at[vsrc[par]], vgs[par],
                                      sms[par]).wait()
                pltpu.make_async_copy(sp_d.at[vdst[par]], vgd[par],
                                      smd[par]).wait()

            def process(par):
                drain(par)
                v_ga, v_gs, v_gd, v_dst = vga[par], vgs[par], vgd[par], vdst[par]

                def vec_group(j0):
                    vsl = pl.ds(j0, 16)
                    e = v_ga[vsl] + v_gd[vsl]
                    e = jnp.where(e >= 0.0, e, 0.2 * e)
                    p = jnp.exp(e - mv)
                    v_p[vsl] = p
                    v_ps[vsl] = p * v_gs[vsl]
                    d = v_dst[vsl]
                    g = (d.astype(jnp.float32)
                         * jnp.float32(1.0 / N_NODES)).astype(jnp.int32)
                    v_dstp[vsl] = d + (NGP - N_NODES) * g

                def vec_body(i, _):
                    for j in range(8):
                        vec_group(i * 128 + j * 16)
                    return 0

                lax.fori_loop(0, CHUNK // 128, vec_body, 0)
                for j in range(CHUNK % 128 // 16):
                    vec_group(CHUNK - CHUNK % 128 + j * 16)

                pltpu.sync_copy(v_p, acc_d.at[v_dstp], add=True)
                pltpu.sync_copy(v_ps, acc_n.at[v_dstp], add=True)

            issue(0, 0)

            def body2(i, _):
                issue(1, 2 * i + 1)
                process(0)
                issue(0, jnp.minimum(2 * i + 2, EPW_CHUNKS - 1))
                process(1)
                return 0

            lax.fori_loop(0, EPW_CHUNKS // 2, body2, 0)
            drain(0)

        do_conv(e1_h, sp_a1, sp_d1, sp_s1, acc_d1, acc_n1, m1)
        do_conv(e2_h, sp_a2, sp_d2, sp_s2, acc_d2, acc_n2, m2)

        plsc.subcore_barrier()
        hsl = pl.ds(c * NG + offa, STRIPE_A)
        for acc, out in ((acc_d1, oden1), (acc_n1, onum1),
                         (acc_d2, oden2), (acc_n2, onum2)):
            pltpu.sync_copy(acc.at[sla], v_stage)
            pltpu.sync_copy(v_stage, out.at[hsl])

    return k(e1, e2, p1f, p2f, mvec1, mvec2, zeros)


def _epilogue_body(d1_ref, n1_ref, d2_ref, n2_ref, out_ref):
    d1 = d1_ref[0] + d1_ref[1]
    n1 = n1_ref[0] + n1_ref[1]
    d2 = d2_ref[0] + d2_ref[1]
    n2 = n2_ref[0] + n2_ref[1]
    score = n1 / (d1 + 1e-16) + n2 / (d2 + 1e-16)
    col = lax.broadcasted_iota(jnp.int32, score.shape, 1)
    score = jnp.where(col < N_NODES, score, -3.4e38)
    mx = jnp.max(score, axis=-1, keepdims=True)
    ex = jnp.exp(score - mx)
    probs = ex / jnp.sum(ex, axis=-1, keepdims=True)
    out_ref[...] = probs[:, :N_NODES]


def _epilogue(den1, num1, den2, num2):
    return pl.pallas_call(
        _epilogue_body,
        out_shape=jax.ShapeDtypeStruct((NUM_GRAPHS, N_NODES), jnp.float32),
    )(den1, num1, den2, num2)


def kernel(x_1, x_2, edge_index_1, edge_index_2,
           W1, a_src1, a_dst1, b1,
           W2, a_src2, a_dst2, b2,
           lin1_W, lin1_b, lin2_W, lin2_b):
    w_eff = (lin1_W @ lin2_W)[:, 0]
    bt_1 = (W1 @ jnp.stack([a_src1, a_dst1, w_eff], axis=1)).T
    bt_2 = (W2 @ jnp.stack([a_src2, a_dst2, w_eff], axis=1)).T

    projt1, pmax1 = _prologue(x_1.T, bt_1)
    projt2, pmax2 = _prologue(x_2.T, bt_2)

    m1 = jnp.max(pmax1[:, 0, 0]) + jnp.max(pmax1[:, 0, 1])
    m2 = jnp.max(pmax2[:, 0, 0]) + jnp.max(pmax2[:, 0, 1])
    mvec1 = jnp.full((16,), m1, jnp.float32)
    mvec2 = jnp.full((16,), m2, jnp.float32)

    e1 = edge_index_1.reshape(-1)
    e2 = edge_index_2.reshape(-1)
    zeros = jnp.zeros((NG,), jnp.float32)

    den1, num1, den2, num2 = _sc_edge_kernel(
        e1, e2, projt1.reshape(-1), projt2.reshape(-1),
        mvec1, mvec2, zeros)

    def _rs(a):
        return a.reshape(2, NUM_GRAPHS, NGP)

    return _epilogue(_rs(den1), _rs(num1), _rs(den2), _rs(num2))


# speedup vs baseline: 1.3707x; 1.3707x over previous
import functools

import jax
import jax.numpy as jnp
from jax import lax
from jax.experimental import pallas as pl
from jax.experimental.pallas import tpu as pltpu
from jax.experimental.pallas import tpu_sc as plsc

N = 100000
N_NODES = 10000
NUM_GRAPHS = 10
E = 3200000
D_IN = 9

NPAD = 100096
NGP = 10240
NG = NUM_GRAPHS * NGP
NW = 32
STRIPE = NPAD // 16
STRIPE_A = NG // 16
CHUNK = 4000
EPW_CHUNKS = E // (NW * CHUNK)

_PLB = 3328
_PGRID = -(-N // _PLB)


def _prologue_body(xt_ref, bt_ref, projt_ref, pmax_ref):
    xt = xt_ref[...]
    bt = bt_ref[...]
    r = lax.dot_general(bt, xt, (((1,), (0,)), ((), ())),
                        preferred_element_type=jnp.float32)
    projt_ref[...] = r
    col = lax.broadcasted_iota(jnp.int32, r.shape, 1) + pl.program_id(0) * _PLB
    rm = jnp.where(col < N, r, -3.4e38)
    pmax_ref[...] = jnp.max(rm, axis=1).reshape(1, 1, 3)


def _prologue(xt, btmat):
    return pl.pallas_call(
        _prologue_body,
        grid=(_PGRID,),
        in_specs=[
            pl.BlockSpec((D_IN, _PLB), lambda g: (0, g)),
            pl.BlockSpec((3, D_IN), lambda g: (0, 0)),
        ],
        out_specs=[
            pl.BlockSpec((3, _PLB), lambda g: (0, g)),
            pl.BlockSpec((1, 1, 3), lambda g: (g, 0, 0)),
        ],
        out_shape=[
            jax.ShapeDtypeStruct((3, NPAD), jnp.float32),
            jax.ShapeDtypeStruct((_PGRID, 1, 3), jnp.float32),
        ],
    )(xt, btmat)


def _sc_edge_kernel(e1, e2, p1f, p2f, mvec1, mvec2, zeros):
    mesh = plsc.VectorSubcoreMesh(core_axis_name="c", subcore_axis_name="s")

    out_type = [jax.ShapeDtypeStruct((2 * NG,), jnp.float32)] * 4
    scratch = [
        pltpu.VMEM_SHARED((NPAD,), jnp.float32),
        pltpu.VMEM_SHARED((NPAD,), jnp.float32),
        pltpu.VMEM_SHARED((NPAD,), jnp.float32),
        pltpu.VMEM_SHARED((NPAD,), jnp.float32),
        pltpu.VMEM_SHARED((NPAD,), jnp.float32),
        pltpu.VMEM_SHARED((NPAD,), jnp.float32),
        pltpu.VMEM_SHARED((NG,), jnp.float32),
        pltpu.VMEM_SHARED((NG,), jnp.float32),
        pltpu.VMEM_SHARED((NG,), jnp.float32),
        pltpu.VMEM_SHARED((NG,), jnp.float32),
        pltpu.VMEM((CHUNK,), jnp.int32),
        pltpu.VMEM((CHUNK,), jnp.int32),
        pltpu.VMEM((CHUNK,), jnp.int32),
        pltpu.VMEM((CHUNK,), jnp.float32),
        pltpu.VMEM((CHUNK,), jnp.float32),
        pltpu.VMEM((CHUNK,), jnp.float32),
        pltpu.VMEM((CHUNK,), jnp.float32),
        pltpu.VMEM((CHUNK,), jnp.float32),
        pltpu.VMEM((16,), jnp.float32),
        pltpu.VMEM((16,), jnp.float32),
        pltpu.VMEM((STRIPE_A,), jnp.float32),
        pltpu.SemaphoreType.DMA,
        pltpu.SemaphoreType.DMA,
        pltpu.SemaphoreType.DMA,
    ]

    @functools.partial(pl.kernel, out_type=out_type, mesh=mesh,
                       scratch_types=scratch)
    def k(e1_h, e2_h, p1_h, p2_h, m1_h, m2_h, zeros_h,
          oden1, onum1, oden2, onum2,
          sp_a1, sp_d1, sp_s1, sp_a2, sp_d2, sp_s2,
          acc_d1, acc_n1, acc_d2, acc_n2,
          v_src, v_dst, v_dstp, v_ga, v_gs, v_gd, v_p, v_ps,
          v_m1, v_m2, v_stage, sem0, sem1, sem2):
        c = lax.axis_index("c")
        s = lax.axis_index("s")
        wid = s * 2 + c
        off = s * STRIPE
        offa = s * STRIPE_A
        sl = pl.ds(off, STRIPE)
        sla = pl.ds(offa, STRIPE_A)

        for t, sp in ((0, sp_a1), (1, sp_d1), (2, sp_s1)):
            pltpu.sync_copy(p1_h.at[pl.ds(t * NPAD + off, STRIPE)],
                            v_stage.at[pl.ds(0, STRIPE)])
            pltpu.sync_copy(v_stage.at[pl.ds(0, STRIPE)], sp.at[sl])
        for t, sp in ((0, sp_a2), (1, sp_d2), (2, sp_s2)):
            pltpu.sync_copy(p2_h.at[pl.ds(t * NPAD + off, STRIPE)],
                            v_stage.at[pl.ds(0, STRIPE)])
            pltpu.sync_copy(v_stage.at[pl.ds(0, STRIPE)], sp.at[sl])
        pltpu.sync_copy(zeros_h.at[sla], v_stage)
        for acc in (acc_d1, acc_n1, acc_d2, acc_n2):
            pltpu.sync_copy(v_stage, acc.at[sla])
        pltpu.sync_copy(m1_h, v_m1)
        pltpu.sync_copy(m2_h, v_m2)
        plsc.subcore_barrier()

        m1 = v_m1[...]
        m2 = v_m2[...]

        def do_conv(e_h, sp_a, sp_d, sp_s, acc_d, acc_n, mv):
            def chunk_body(kk, _):
                base = (wid * EPW_CHUNKS + kk) * CHUNK
                pltpu.sync_copy(e_h.at[pl.ds(base, CHUNK)], v_src)
                pltpu.sync_copy(e_h.at[pl.ds(E + base, CHUNK)], v_dst)
                cp0 = pltpu.async_copy(sp_a.at[v_src], v_ga, sem0)
                cp1 = pltpu.async_copy(sp_s.at[v_src], v_gs, sem1)
                cp2 = pltpu.async_copy(sp_d.at[v_dst], v_gd, sem2)
                cp0.wait()
                cp1.wait()
                cp2.wait()

                def vec_group(j0):
                    vsl = pl.ds(j0, 16)
                    e = v_ga[vsl] + v_gd[vsl]
                    e = jnp.where(e >= 0.0, e, 0.2 * e)
                    p = jnp.exp(e - mv)
                    v_p[vsl] = p
                    v_ps[vsl] = p * v_gs[vsl]
                    d = v_dst[vsl]
                    g = (d.astype(jnp.float32)
                         * jnp.float32(1.0 / N_NODES)).astype(jnp.int32)
                    v_dstp[vsl] = d + (NGP - N_NODES) * g

                def vec_body(i, _):
                    for j in range(8):
                        vec_group(i * 128 + j * 16)
                    return 0

                lax.fori_loop(0, CHUNK // 128, vec_body, 0)
                for j in range(CHUNK % 128 // 16):
                    vec_group(CHUNK - CHUNK % 128 + j * 16)

                pltpu.sync_copy(v_p, acc_d.at[v_dstp], add=True)
                pltpu.sync_copy(v_ps, acc_n.at[v_dstp], add=True)
                return 0

            lax.fori_loop(0, EPW_CHUNKS, chunk_body, 0)

        do_conv(e1_h, sp_a1, sp_d1, sp_s1, acc_d1, acc_n1, m1)
        do_conv(e2_h, sp_a2, sp_d2, sp_s2, acc_d2, acc_n2, m2)

        plsc.subcore_barrier()
        hsl = pl.ds(c * NG + offa, STRIPE_A)
        for acc, out in ((acc_d1, oden1), (acc_n1, onum1),
                         (acc_d2, oden2), (acc_n2, onum2)):
            pltpu.sync_copy(acc.at[sla], v_stage)
            pltpu.sync_copy(v_stage, out.at[hsl])

    return k(e1, e2, p1f, p2f, mvec1, mvec2, zeros)


def _epilogue_body(d1_ref, n1_ref, d2_ref, n2_ref, out_ref):
    d1 = d1_ref[0] + d1_ref[1]
    n1 = n1_ref[0] + n1_ref[1]
    d2 = d2_ref[0] + d2_ref[1]
    n2 = n2_ref[0] + n2_ref[1]
    score = n1 / (d1 + 1e-16) + n2 / (d2 + 1e-16)
    col = lax.broadcasted_iota(jnp.int32, score.shape, 1)
    score = jnp.where(col < N_NODES, score, -3.4e38)
    mx = jnp.max(score, axis=-1, keepdims=True)
    ex = jnp.exp(score - mx)
    probs = ex / jnp.sum(ex, axis=-1, keepdims=True)
    out_ref[...] = probs[:, :N_NODES]


def _epilogue(den1, num1, den2, num2):
    return pl.pallas_call(
        _epilogue_body,
        out_shape=jax.ShapeDtypeStruct((NUM_GRAPHS, N_NODES), jnp.float32),
    )(den1, num1, den2, num2)


def kernel(x_1, x_2, edge_index_1, edge_index_2,
           W1, a_src1, a_dst1, b1,
           W2, a_src2, a_dst2, b2,
           lin1_W, lin1_b, lin2_W, lin2_b):
    w_eff = (lin1_W @ lin2_W)[:, 0]
    bt_1 = (W1 @ jnp.stack([a_src1, a_dst1, w_eff], axis=1)).T
    bt_2 = (W2 @ jnp.stack([a_src2, a_dst2, w_eff], axis=1)).T

    projt1, pmax1 = _prologue(x_1.T, bt_1)
    projt2, pmax2 = _prologue(x_2.T, bt_2)

    m1 = jnp.max(pmax1[:, 0, 0]) + jnp.max(pmax1[:, 0, 1])
    m2 = jnp.max(pmax2[:, 0, 0]) + jnp.max(pmax2[:, 0, 1])
    mvec1 = jnp.full((16,), m1, jnp.float32)
    mvec2 = jnp.full((16,), m2, jnp.float32)

    e1 = edge_index_1.reshape(-1)
    e2 = edge_index_2.reshape(-1)
    zeros = jnp.zeros((NG,), jnp.float32)

    den1, num1, den2, num2 = _sc_edge_kernel(
        e1, e2, projt1.reshape(-1), projt2.reshape(-1),
        mvec1, mvec2, zeros)

    def _rs(a):
        return a.reshape(2, NUM_GRAPHS, NGP)

    return _epilogue(_rs(den1), _rs(num1), _rs(den2), _rs(num2))
